# baseline (device time: 102245 ns/iter reference)
import jax
import jax.numpy as jnp
from jax import lax
from jax.experimental import pallas as pl
from jax.experimental.pallas import tpu as pltpu

N_DEV = 4
B = 2
SQ = 512
SKV = 512
HG = 8
DH = 64
D_MODEL = 768
BLK = 64


def kernel(x, Wq, K_ext, V_ext, Wo):
    K_r = jnp.transpose(
        K_ext.reshape(B, SKV, N_DEV, HG, DH), (2, 0, 3, 1, 4)
    ).astype(jnp.bfloat16)
    V_r = jnp.transpose(
        V_ext.reshape(B, SKV, N_DEV, HG, DH), (2, 0, 3, 1, 4)
    ).astype(jnp.bfloat16)
    V_aug = jnp.concatenate(
        [V_r, jnp.ones((N_DEV, B, HG, SKV, 1), jnp.bfloat16)], axis=-1
    )
    x16 = (x * 0.125).astype(jnp.bfloat16)
    W_pair = jnp.stack(
        [Wq.astype(jnp.bfloat16), Wo.T.astype(jnp.bfloat16)], axis=0
    )

    def body(x_ref, w_ref, k_ref, v_ref, out_ref, comm, send_sems, recv_sems):
        my = lax.axis_index("i")
        left = (my + N_DEV - 1) % N_DEV
        right = (my + 1) % N_DEV

        barrier_sem = pltpu.get_barrier_semaphore()
        for nbr in (left, right):
            pl.semaphore_signal(
                barrier_sem, inc=1,
                device_id=(nbr,), device_id_type=pl.DeviceIdType.MESH,
            )
        pl.semaphore_wait(barrier_sem, 2)

        comm[0] = w_ref[...]

        ri = lax.broadcasted_iota(jnp.int32, (SQ, SKV), 0)
        ci = lax.broadcasted_iota(jnp.int32, (SQ, SKV), 1)
        qb = my * (SQ // BLK) + ri // BLK
        kb = ci // BLK
        mask = (qb == kb) | (kb == 0) | ((qb + kb) % 3 == 0)
        bias = jnp.where(mask, 0.0, -30.0).astype(jnp.float32)

        x2 = x_ref[...].reshape(B * SQ, D_MODEL)

        def contribution(slot, origin):
            wq = comm[slot, 0]
            wot = comm[slot, 1]
            q_all = jnp.dot(
                x2, wq, preferred_element_type=jnp.float32
            ).astype(jnp.bfloat16)
            for b in range(B):
                q = q_all[b * SQ:(b + 1) * SQ]
                ctx = []
                for hh in range(HG):
                    qh = q[:, hh * DH:(hh + 1) * DH]
                    kh = k_ref[origin, b, hh]
                    va = v_ref[origin, b, hh]
                    s = lax.dot_general(
                        qh, kh, (((1,), (1,)), ((), ())),
                        preferred_element_type=jnp.float32,
                    )
                    e = jnp.exp(s + bias).astype(jnp.bfloat16)
                    cs = jnp.dot(e, va, preferred_element_type=jnp.float32)
                    inv = 1.0 / cs[:, DH:DH + 1]
                    ctx.append((cs[:, :DH] * inv).astype(jnp.bfloat16))
                ctx = jnp.concatenate(ctx, axis=1)
                part = lax.dot_general(
                    ctx, wot, (((1,), (1,)), ((), ())),
                    preferred_element_type=jnp.float32,
                )
                if slot == 0:
                    out_ref[b] = part
                else:
                    out_ref[b] = out_ref[b] + part

        for h in range(N_DEV - 1):
            rdma = pltpu.make_async_remote_copy(
                src_ref=comm.at[h], dst_ref=comm.at[h + 1],
                send_sem=send_sems.at[h], recv_sem=recv_sems.at[h],
                device_id=(right,), device_id_type=pl.DeviceIdType.MESH,
            )
            rdma.start()
            contribution(h, (my + N_DEV - h) % N_DEV)
            rdma.wait()

        contribution(N_DEV - 1, (my + 1) % N_DEV)

    return pl.pallas_call(
        body,
        out_shape=jax.ShapeDtypeStruct((B, SQ, D_MODEL), jnp.float32),
        in_specs=[pl.BlockSpec(memory_space=pltpu.VMEM)] * 4,
        out_specs=pl.BlockSpec(memory_space=pltpu.VMEM),
        scratch_shapes=[
            pltpu.VMEM((N_DEV, 2, D_MODEL, HG * DH), jnp.bfloat16),
            pltpu.SemaphoreType.DMA((N_DEV - 1,)),
            pltpu.SemaphoreType.DMA((N_DEV - 1,)),
        ],
        compiler_params=pltpu.CompilerParams(
            collective_id=0, vmem_limit_bytes=100 * 1024 * 1024,
        ),
    )(x16, W_pair, K_r, V_aug)


# device time: 78623 ns/iter; 1.3004x vs baseline; 1.3004x over previous
import jax
import jax.numpy as jnp
from jax import lax
from jax.experimental import pallas as pl
from jax.experimental.pallas import tpu as pltpu

N_DEV = 4
B = 2
SQ = 512
SKV = 512
HG = 8
HH = HG // 2
DH = 64
D_MODEL = 768
BLK = 64


def kernel(x, Wq, K_ext, V_ext, Wo):
    K_r = jnp.transpose(
        K_ext.reshape(B, SKV, N_DEV, HG, DH), (2, 0, 3, 1, 4)
    ).astype(jnp.bfloat16)
    V_r = jnp.transpose(
        V_ext.reshape(B, SKV, N_DEV, HG, DH), (2, 0, 3, 1, 4)
    ).astype(jnp.bfloat16)
    V_aug = jnp.concatenate(
        [V_r, jnp.ones((N_DEV, B, HG, SKV, 1), jnp.bfloat16)], axis=-1
    )
    x16 = (x * 0.125).astype(jnp.bfloat16)
    Wq16 = Wq.astype(jnp.bfloat16)
    WoT16 = Wo.T.astype(jnp.bfloat16)
    HW = HH * DH
    W_right = jnp.stack([Wq16[:, :HW], WoT16[:, :HW]], axis=0)
    W_left = jnp.stack([Wq16[:, HW:], WoT16[:, HW:]], axis=0)

    def body(x_ref, wr_ref, wl_ref, k_ref, v_ref, out_ref,
             comm_r, comm_l, send_r, recv_r, send_l, recv_l):
        my = lax.axis_index("i")
        left = (my + N_DEV - 1) % N_DEV
        right = (my + 1) % N_DEV

        barrier_sem = pltpu.get_barrier_semaphore()
        for nbr in (left, right):
            pl.semaphore_signal(
                barrier_sem, inc=1,
                device_id=(nbr,), device_id_type=pl.DeviceIdType.MESH,
            )
        pl.semaphore_wait(barrier_sem, 2)

        comm_r[0] = wr_ref[...]
        comm_l[0] = wl_ref[...]

        ri = lax.broadcasted_iota(jnp.int32, (SQ, SKV), 0)
        ci = lax.broadcasted_iota(jnp.int32, (SQ, SKV), 1)
        qb = my * (SQ // BLK) + ri // BLK
        kb = ci // BLK
        mask = (qb == kb) | (kb == 0) | ((qb + kb) % 3 == 0)
        bias = jnp.where(mask, 0.0, -30.0).astype(jnp.float32)

        x2 = x_ref[...].reshape(B * SQ, D_MODEL)

        def contribution(buf, slot, origin, head_off, init):
            wq = buf[slot, 0]
            wot = buf[slot, 1]
            q_all = jnp.dot(
                x2, wq, preferred_element_type=jnp.float32
            ).astype(jnp.bfloat16)
            for b in range(B):
                q = q_all[b * SQ:(b + 1) * SQ]
                ctx = []
                for hh in range(HH):
                    qh = q[:, hh * DH:(hh + 1) * DH]
                    kh = k_ref[origin, b, head_off + hh]
                    va = v_ref[origin, b, head_off + hh]
                    s = lax.dot_general(
                        qh, kh, (((1,), (1,)), ((), ())),
                        preferred_element_type=jnp.float32,
                    )
                    e = jnp.exp(s + bias).astype(jnp.bfloat16)
                    cs = jnp.dot(e, va, preferred_element_type=jnp.float32)
                    inv = 1.0 / cs[:, DH:DH + 1]
                    ctx.append((cs[:, :DH] * inv).astype(jnp.bfloat16))
                ctx = jnp.concatenate(ctx, axis=1)
                part = lax.dot_general(
                    ctx, wot, (((1,), (1,)), ((), ())),
                    preferred_element_type=jnp.float32,
                )
                if init:
                    out_ref[b] = part
                else:
                    out_ref[b] = out_ref[b] + part

        for h in range(N_DEV - 1):
            rr = pltpu.make_async_remote_copy(
                src_ref=comm_r.at[h], dst_ref=comm_r.at[h + 1],
                send_sem=send_r.at[h], recv_sem=recv_r.at[h],
                device_id=(right,), device_id_type=pl.DeviceIdType.MESH,
            )
            rl = pltpu.make_async_remote_copy(
                src_ref=comm_l.at[h], dst_ref=comm_l.at[h + 1],
                send_sem=send_l.at[h], recv_sem=recv_l.at[h],
                device_id=(left,), device_id_type=pl.DeviceIdType.MESH,
            )
            rr.start()
            rl.start()
            contribution(comm_r, h, (my + N_DEV - h) % N_DEV, 0, h == 0)
            contribution(comm_l, h, (my + h) % N_DEV, HH, False)
            rr.wait()
            rl.wait()

        contribution(comm_r, N_DEV - 1, (my + 1) % N_DEV, 0, False)
        contribution(comm_l, N_DEV - 1, (my + N_DEV - 1) % N_DEV, HH, False)

    return pl.pallas_call(
        body,
        out_shape=jax.ShapeDtypeStruct((B, SQ, D_MODEL), jnp.float32),
        in_specs=[pl.BlockSpec(memory_space=pltpu.VMEM)] * 5,
        out_specs=pl.BlockSpec(memory_space=pltpu.VMEM),
        scratch_shapes=[
            pltpu.VMEM((N_DEV, 2, D_MODEL, HW), jnp.bfloat16),
            pltpu.VMEM((N_DEV, 2, D_MODEL, HW), jnp.bfloat16),
            pltpu.SemaphoreType.DMA((N_DEV - 1,)),
            pltpu.SemaphoreType.DMA((N_DEV - 1,)),
            pltpu.SemaphoreType.DMA((N_DEV - 1,)),
            pltpu.SemaphoreType.DMA((N_DEV - 1,)),
        ],
        compiler_params=pltpu.CompilerParams(
            collective_id=0, vmem_limit_bytes=100 * 1024 * 1024,
        ),
    )(x16, W_right, W_left, K_r, V_aug)


# device time: 77664 ns/iter; 1.3165x vs baseline; 1.0123x over previous
import jax
import jax.numpy as jnp
from jax import lax
from jax.experimental import pallas as pl
from jax.experimental.pallas import tpu as pltpu

N_DEV = 4
B = 2
SQ = 512
SKV = 512
HG = 8
HH = HG // 2
DH = 64
D_MODEL = 768
BLK = 64


def kernel(x, Wq, K_ext, V_ext, Wo):
    K_r = jnp.transpose(
        K_ext.reshape(B, SKV, N_DEV, HG, DH), (2, 0, 3, 1, 4)
    ).astype(jnp.bfloat16)
    V_r = jnp.transpose(
        V_ext.reshape(B, SKV, N_DEV, HG, DH), (2, 0, 3, 1, 4)
    ).astype(jnp.bfloat16)
    V_aug = jnp.concatenate(
        [V_r, jnp.ones((N_DEV, B, HG, SKV, 1), jnp.bfloat16)], axis=-1
    )
    x16 = (x * 0.125).astype(jnp.bfloat16)
    Wq16 = Wq.astype(jnp.bfloat16)
    WoT16 = Wo.T.astype(jnp.bfloat16)
    HW = HH * DH
    W_right = jnp.stack([Wq16[:, :HW], WoT16[:, :HW]], axis=0)
    W_left = jnp.stack([Wq16[:, HW:], WoT16[:, HW:]], axis=0)

    def body(x_ref, wr_ref, wl_ref, k_ref, v_ref, out_ref,
             comm_r, comm_l, send_r, recv_r, send_l, recv_l):
        my = lax.axis_index("i")
        left = (my + N_DEV - 1) % N_DEV
        right = (my + 1) % N_DEV

        barrier_sem = pltpu.get_barrier_semaphore()
        for nbr in (left, right):
            pl.semaphore_signal(
                barrier_sem, inc=1,
                device_id=(nbr,), device_id_type=pl.DeviceIdType.MESH,
            )
        pl.semaphore_wait(barrier_sem, 2)

        comm_r[0] = wr_ref[...]
        comm_l[0] = wl_ref[...]

        ri = lax.broadcasted_iota(jnp.int32, (SQ, SKV), 0)
        ci = lax.broadcasted_iota(jnp.int32, (SQ, SKV), 1)
        qb = my * (SQ // BLK) + ri // BLK
        kb = ci // BLK
        mask = (qb == kb) | (kb == 0) | ((qb + kb) % 3 == 0)
        bias = jnp.where(mask, 0.0, -30.0).astype(jnp.float32)

        x2 = x_ref[...].reshape(B * SQ, D_MODEL)

        def contribution(buf, slot, origin, head_off, init):
            wq = buf[slot, 0]
            wot = buf[slot, 1]
            q_all = jnp.dot(
                x2, wq, preferred_element_type=jnp.float32
            ).astype(jnp.bfloat16)
            for b in range(B):
                q = q_all[b * SQ:(b + 1) * SQ]
                ctx = []
                for hh in range(HH):
                    qh = q[:, hh * DH:(hh + 1) * DH]
                    kh = k_ref[origin, b, head_off + hh]
                    va = v_ref[origin, b, head_off + hh]
                    s = lax.dot_general(
                        qh, kh, (((1,), (1,)), ((), ())),
                        preferred_element_type=jnp.float32,
                    )
                    e = jnp.exp(s + bias).astype(jnp.bfloat16)
                    cs = jnp.dot(e, va, preferred_element_type=jnp.float32)
                    inv = 1.0 / cs[:, DH:DH + 1]
                    ctx.append((cs[:, :DH] * inv).astype(jnp.bfloat16))
                ctx = jnp.concatenate(ctx, axis=1)
                part = lax.dot_general(
                    ctx, wot, (((1,), (1,)), ((), ())),
                    preferred_element_type=jnp.float32,
                )
                if init:
                    out_ref[b] = part
                else:
                    out_ref[b] = out_ref[b] + part

        def mk(buf, h, c, ssem, rsem, dev):
            return pltpu.make_async_remote_copy(
                src_ref=buf.at[h, c], dst_ref=buf.at[h + 1, c],
                send_sem=ssem.at[h, c], recv_sem=rsem.at[h, c],
                device_id=(dev,), device_id_type=pl.DeviceIdType.MESH,
            )

        rdma = [
            [[mk(comm_r, h, c, send_r, recv_r, right) for c in range(2)],
             [mk(comm_l, h, c, send_l, recv_l, left) for c in range(2)]]
            for h in range(N_DEV - 1)
        ]

        for d in range(2):
            for c in range(2):
                rdma[0][d][c].start()
        contribution(comm_r, 0, my, 0, True)
        contribution(comm_l, 0, my, HH, False)

        for h in range(N_DEV - 2):
            for d in range(2):
                for c in range(2):
                    rdma[h][d][c].wait_recv()
                    rdma[h + 1][d][c].start()
            contribution(comm_r, h + 1, (my + N_DEV - h - 1) % N_DEV, 0,
                         False)
            contribution(comm_l, h + 1, (my + h + 1) % N_DEV, HH, False)

        for d in range(2):
            for c in range(2):
                rdma[N_DEV - 2][d][c].wait_recv()
        contribution(comm_r, N_DEV - 1, (my + 1) % N_DEV, 0, False)
        contribution(comm_l, N_DEV - 1, (my + N_DEV - 1) % N_DEV, HH, False)

        for h in range(N_DEV - 1):
            for d in range(2):
                for c in range(2):
                    rdma[h][d][c].wait_send()

    return pl.pallas_call(
        body,
        out_shape=jax.ShapeDtypeStruct((B, SQ, D_MODEL), jnp.float32),
        in_specs=[pl.BlockSpec(memory_space=pltpu.VMEM)] * 5,
        out_specs=pl.BlockSpec(memory_space=pltpu.VMEM),
        scratch_shapes=[
            pltpu.VMEM((N_DEV, 2, D_MODEL, HW), jnp.bfloat16),
            pltpu.VMEM((N_DEV, 2, D_MODEL, HW), jnp.bfloat16),
            pltpu.SemaphoreType.DMA((N_DEV - 1, 2)),
            pltpu.SemaphoreType.DMA((N_DEV - 1, 2)),
            pltpu.SemaphoreType.DMA((N_DEV - 1, 2)),
            pltpu.SemaphoreType.DMA((N_DEV - 1, 2)),
        ],
        compiler_params=pltpu.CompilerParams(
            collective_id=0, vmem_limit_bytes=100 * 1024 * 1024,
        ),
    )(x16, W_right, W_left, K_r, V_aug)
